# baseline (speedup 1.0000x reference)
import functools

import jax
import jax.numpy as jnp
from jax import lax
from jax.experimental import pallas as pl
from jax.experimental.pallas import tpu as pltpu
from jax.experimental.pallas import tpu_sc as plsc

B = 16384
DEG = 32
D = 128
LANES = 16
NW = 32
NB_W = B // NW
CHUNK = 64
ITEMS_PER_CHUNK = CHUNK // DEG
CHUNKS_W = NB_W * DEG // CHUNK
N_STAGE = NB_W // CHUNK
CHUNKS_PER_STAGE = CHUNKS_W // N_STAGE


NBUF = 8


def _sc_body(nodes_hbm, neigh_hbm, feat_hbm, self_out, neigh_out, *scr):
    nidx_v, sidx_v = scr[0], scr[1]
    rows_bufs = scr[2:2 + NBUF]
    srows_v = scr[2 + NBUF]
    stage_v = scr[3 + NBUF]
    sems = scr[4 + NBUF:4 + 2 * NBUF]
    semself = scr[4 + 2 * NBUF]
    semf = (scr[5 + 2 * NBUF], scr[6 + 2 * NBUF])
    wid = lax.axis_index("s") * 2 + lax.axis_index("c")
    pltpu.sync_copy(neigh_hbm.at[pl.ds(wid * CHUNKS_W, CHUNKS_W)], nidx_v)
    pltpu.sync_copy(nodes_hbm.at[pl.ds(wid * N_STAGE, N_STAGE)], sidx_v)
    out_base = wid * NB_W

    def start(c, rows, sem):
        pltpu.async_copy(feat_hbm.at[nidx_v.at[c]], rows, sem)

    def finish(c, rows, sem):
        pltpu.make_async_copy(feat_hbm.at[nidx_v.at[c]], rows, sem).wait()

    def row_vals(rows, r):
        return tuple(rows[r, pl.ds(d * LANES, LANES)]
                     for d in range(D // LANES))

    def reduce_chunk(rows, stage_base):
        for i in range(ITEMS_PER_CHUNK):
            base_r = i * DEG
            v0 = row_vals(rows, base_r)
            v1 = row_vals(rows, base_r + 1)
            accs = tuple(v0[d] + v1[d] for d in range(D // LANES))

            def add_body(k2, acc, base_r=base_r, rows=rows):
                r = base_r + 2 * k2
                va = row_vals(rows, r)
                vb = row_vals(rows, r + 1)
                return tuple(acc[d] + (va[d] + vb[d])
                             for d in range(D // LANES))

            accs = lax.fori_loop(1, DEG // 2, add_body, accs)
            for d in range(D // LANES):
                stage_v[stage_base + i, pl.ds(d * LANES, LANES)] = accs[d]

    for b in range(NBUF):
        start(b, rows_bufs[b], sems[b])

    grp = CHUNKS_PER_STAGE // NBUF

    def flush_desc(half, q, sem):
        return pltpu.make_async_copy(
            stage_v.at[pl.ds(half * CHUNK, CHUNK)],
            neigh_out.at[pl.ds(out_base + q * CHUNK, CHUNK)],
            sem)

    def stage_body(q, _):
        par = lax.rem(q, 2)
        half_off = par * CHUNK
        pltpu.async_copy(feat_hbm.at[sidx_v.at[q]], srows_v, semself)

        @pl.when(q >= 2)
        def _():
            @pl.when(par == 0)
            def _():
                flush_desc(par, q - 2, semf[0]).wait()

            @pl.when(par == 1)
            def _():
                flush_desc(par, q - 2, semf[1]).wait()

        def group_body(j, _):
            local = j - q * grp
            for b in range(NBUF):
                c = NBUF * j + b
                finish(c, rows_bufs[b], sems[b])
                reduce_chunk(
                    rows_bufs[b],
                    half_off + (local * NBUF + b) * ITEMS_PER_CHUNK)

                @pl.when(c + NBUF < CHUNKS_W)
                def _(c=c, b=b):
                    start(c + NBUF, rows_bufs[b], sems[b])

            return 0

        lax.fori_loop(q * grp, (q + 1) * grp, group_body, 0)
        pltpu.make_async_copy(feat_hbm.at[sidx_v.at[q]], srows_v, semself).wait()
        pltpu.sync_copy(srows_v, self_out.at[pl.ds(out_base + q * CHUNK, CHUNK)])

        @pl.when(par == 0)
        def _():
            flush_desc(par, q, semf[0]).start()

        @pl.when(par == 1)
        def _():
            flush_desc(par, q, semf[1]).start()

        return 0

    lax.fori_loop(0, N_STAGE, stage_body, 0)
    flush_desc(0, N_STAGE - 2, semf[0]).wait()
    flush_desc(1, N_STAGE - 1, semf[1]).wait()


_sc_gather_mean = functools.partial(
    pl.kernel,
    out_type=(jax.ShapeDtypeStruct((B, D), jnp.float32),
              jax.ShapeDtypeStruct((B, D), jnp.float32)),
    mesh=plsc.VectorSubcoreMesh(core_axis_name="c", subcore_axis_name="s"),
    scratch_types=(
        [pltpu.VMEM((CHUNKS_W, CHUNK), jnp.int32),
         pltpu.VMEM((N_STAGE, CHUNK), jnp.int32)]
        + [pltpu.VMEM((CHUNK, D), jnp.float32)
           for _ in range(NBUF)]
        + [pltpu.VMEM((CHUNK, D), jnp.float32),
           pltpu.VMEM((2 * CHUNK, D), jnp.float32)]
        + [pltpu.SemaphoreType.DMA for _ in range(NBUF + 3)]
    ),
)(_sc_body)


def _tc_head(self_feats, neigh_sum, Wt, Wb_perm, b1):
    BB = 2048

    def mm(self_ref, neigh_ref, wt_ref, wb_ref, b_ref, o_ref):
        s = self_ref[...]
        n = neigh_ref[...] * (1.0 / DEG)
        y = jnp.dot(s, wt_ref[...], preferred_element_type=jnp.float32)
        y = y + jnp.dot(n, wb_ref[...], preferred_element_type=jnp.float32)
        y = y + b_ref[...]
        o_ref[...] = jnp.maximum(y, 0.0)

    return pl.pallas_call(
        mm,
        grid=(B // BB,),
        in_specs=[
            pl.BlockSpec((BB, D), lambda i: (i, 0)),
            pl.BlockSpec((BB, D), lambda i: (i, 0)),
            pl.BlockSpec((D, D), lambda i: (0, 0)),
            pl.BlockSpec((D, D), lambda i: (0, 0)),
            pl.BlockSpec((1, D), lambda i: (0, 0)),
        ],
        out_specs=pl.BlockSpec((BB, D), lambda i: (i, 0)),
        out_shape=jax.ShapeDtypeStruct((B, D), jnp.float32),
    )(self_feats, neigh_sum, Wt, Wb_perm, b1.reshape(1, D))


def kernel(nodes, neigh_index, features, W1, b1):
    nodes2d = nodes.astype(jnp.int32).reshape(B // CHUNK, CHUNK)
    neigh2d = neigh_index.astype(jnp.int32).reshape(B * DEG // CHUNK, CHUNK)
    self_feats, neigh_sum = _sc_gather_mean(nodes2d, neigh2d, features)
    return _tc_head(self_feats, neigh_sum, W1[:D], W1[D:], b1)


# speedup vs baseline: 1.0056x; 1.0056x over previous
import functools

import jax
import jax.numpy as jnp
from jax import lax
from jax.experimental import pallas as pl
from jax.experimental.pallas import tpu as pltpu
from jax.experimental.pallas import tpu_sc as plsc

B = 16384
DEG = 32
D = 128
LANES = 16
NW = 32
NB_W = B // NW
CHUNK = 64
ITEMS_PER_CHUNK = CHUNK // DEG
CHUNKS_W = NB_W * DEG // CHUNK
N_STAGE = NB_W // CHUNK
CHUNKS_PER_STAGE = CHUNKS_W // N_STAGE


NBUF = 8


def _sc_body(nodes_hbm, neigh_hbm, feat_hbm, self_out, neigh_out, *scr):
    nidx_v, sidx_v = scr[0], scr[1]
    rows_bufs = scr[2:2 + NBUF]
    srows_v = scr[2 + NBUF]
    stage_v = scr[3 + NBUF]
    sems = scr[4 + NBUF:4 + 2 * NBUF]
    semself = scr[4 + 2 * NBUF]
    wid = lax.axis_index("s") * 2 + lax.axis_index("c")
    pltpu.sync_copy(neigh_hbm.at[pl.ds(wid * CHUNKS_W, CHUNKS_W)], nidx_v)
    pltpu.sync_copy(nodes_hbm.at[pl.ds(wid * N_STAGE, N_STAGE)], sidx_v)
    out_base = wid * NB_W

    def start(c, rows, sem):
        pltpu.async_copy(feat_hbm.at[nidx_v.at[c]], rows, sem)

    def finish(c, rows, sem):
        pltpu.make_async_copy(feat_hbm.at[nidx_v.at[c]], rows, sem).wait()

    def row_vals(rows, r):
        return tuple(rows[r, pl.ds(d * LANES, LANES)]
                     for d in range(D // LANES))

    def reduce_chunk(rows, stage_base):
        for i in range(ITEMS_PER_CHUNK):
            base_r = i * DEG
            v0 = row_vals(rows, base_r)
            v1 = row_vals(rows, base_r + 1)
            accs = tuple(v0[d] + v1[d] for d in range(D // LANES))

            def add_body(k2, acc, base_r=base_r, rows=rows):
                r = base_r + 2 * k2
                va = row_vals(rows, r)
                vb = row_vals(rows, r + 1)
                return tuple(acc[d] + (va[d] + vb[d])
                             for d in range(D // LANES))

            accs = lax.fori_loop(1, DEG // 2, add_body, accs)
            for d in range(D // LANES):
                stage_v[stage_base + i, pl.ds(d * LANES, LANES)] = accs[d]

    for b in range(NBUF):
        start(b, rows_bufs[b], sems[b])

    grp = CHUNKS_PER_STAGE // NBUF

    def stage_body(q, _):
        pltpu.async_copy(feat_hbm.at[sidx_v.at[q]], srows_v, semself)

        def group_body(j, _):
            local = j - q * grp
            for b in range(NBUF):
                c = NBUF * j + b
                finish(c, rows_bufs[b], sems[b])
                reduce_chunk(rows_bufs[b],
                             (local * NBUF + b) * ITEMS_PER_CHUNK)

                @pl.when(c + NBUF < CHUNKS_W)
                def _(c=c, b=b):
                    start(c + NBUF, rows_bufs[b], sems[b])

            return 0

        lax.fori_loop(q * grp, (q + 1) * grp, group_body, 0)
        pltpu.make_async_copy(feat_hbm.at[sidx_v.at[q]], srows_v, semself).wait()
        pltpu.sync_copy(srows_v, self_out.at[pl.ds(out_base + q * CHUNK, CHUNK)])
        pltpu.sync_copy(stage_v, neigh_out.at[pl.ds(out_base + q * CHUNK, CHUNK)])
        return 0

    lax.fori_loop(0, N_STAGE, stage_body, 0)


_sc_gather_mean = functools.partial(
    pl.kernel,
    out_type=(jax.ShapeDtypeStruct((B, D), jnp.float32),
              jax.ShapeDtypeStruct((B, D), jnp.float32)),
    mesh=plsc.VectorSubcoreMesh(core_axis_name="c", subcore_axis_name="s"),
    scratch_types=(
        [pltpu.VMEM((CHUNKS_W, CHUNK), jnp.int32),
         pltpu.VMEM((N_STAGE, CHUNK), jnp.int32)]
        + [pltpu.VMEM((CHUNK, D), jnp.float32)
           for _ in range(NBUF)]
        + [pltpu.VMEM((CHUNK, D), jnp.float32),
           pltpu.VMEM((CHUNK, D), jnp.float32)]
        + [pltpu.SemaphoreType.DMA for _ in range(NBUF + 1)]
    ),
)(_sc_body)


def _tc_head(self_feats, neigh_sum, Wt, Wb, b1):
    BB = 2048

    def mm(self_ref, neigh_ref, wt_ref, wb_ref, b_ref, o_ref):
        s = self_ref[...]
        n = neigh_ref[...] * (1.0 / DEG)
        y = jnp.dot(s, wt_ref[...], preferred_element_type=jnp.float32)
        y = y + jnp.dot(n, wb_ref[...], preferred_element_type=jnp.float32)
        y = y + b_ref[...]
        o_ref[...] = jnp.maximum(y, 0.0)

    return pl.pallas_call(
        mm,
        grid=(B // BB,),
        in_specs=[
            pl.BlockSpec((BB, D), lambda i: (i, 0)),
            pl.BlockSpec((BB, D), lambda i: (i, 0)),
            pl.BlockSpec((D, D), lambda i: (0, 0)),
            pl.BlockSpec((D, D), lambda i: (0, 0)),
            pl.BlockSpec((1, D), lambda i: (0, 0)),
        ],
        out_specs=pl.BlockSpec((BB, D), lambda i: (i, 0)),
        out_shape=jax.ShapeDtypeStruct((B, D), jnp.float32),
    )(self_feats, neigh_sum, Wt, Wb, b1.reshape(1, D))


def kernel(nodes, neigh_index, features, W1, b1):
    nodes2d = nodes.astype(jnp.int32).reshape(B // CHUNK, CHUNK)
    neigh2d = neigh_index.astype(jnp.int32).reshape(B * DEG // CHUNK, CHUNK)
    self_feats, neigh_sum = _sc_gather_mean(nodes2d, neigh2d, features)
    return _tc_head(self_feats, neigh_sum, W1[:D], W1[D:], b1)
